# baseline (speedup 1.0000x reference)
import functools

import jax
import jax.numpy as jnp
from jax import lax
from jax.experimental import pallas as pl
from jax.experimental.pallas import tpu as pltpu
from jax.experimental.pallas import tpu_sc as plsc

N = 16384
NS = 16
CHUNK = N // NS
L = 16
VECS = CHUNK // L

_mesh = plsc.VectorSubcoreMesh(
    core_axis_name="c", subcore_axis_name="s", num_cores=1, num_subcores=NS
)


@functools.partial(
    pl.kernel,
    out_type=jax.ShapeDtypeStruct((L,), jnp.float32),
    mesh=_mesh,
    compiler_params=pltpu.CompilerParams(
        needs_layout_passes=False, skip_device_barrier=True
    ),
    scratch_types=[
        pltpu.VMEM((CHUNK,), jnp.float32),
        pltpu.VMEM((CHUNK,), jnp.int32),
        pltpu.VMEM((CHUNK,), jnp.float32),
        pltpu.VMEM((2, L), jnp.float32),
        pltpu.VMEM((NS, 2, L), jnp.float32),
        pltpu.VMEM((L,), jnp.float32),
        pltpu.VMEM_SHARED((NS, 2, L), jnp.float32),
        pltpu.SemaphoreType.DMA,
    ],
)
def _masked_mse_sc(predict_hbm, events_hbm, rfs_hbm, out_hbm,
                   p_v, e_v, t_v, part_v, all_v, res_v, shared, sem):
    wid = lax.axis_index("s")
    base = wid * CHUNK

    c1 = pltpu.make_async_copy(predict_hbm.at[pl.ds(base, CHUNK)], p_v, sem)
    c2 = pltpu.make_async_copy(events_hbm.at[pl.ds(base, CHUNK)], e_v, sem)
    c3 = pltpu.make_async_copy(rfs_hbm.at[pl.ds(base, CHUNK)], t_v, sem)
    c1.start()
    c2.start()
    c3.start()
    c1.wait()
    c2.wait()
    c3.wait()

    acc_sq = jnp.zeros((L,), jnp.float32)
    acc_ct = jnp.zeros((L,), jnp.float32)
    for i in range(VECS):
        sl = pl.ds(i * L, L)
        m = e_v[sl].astype(jnp.float32)
        d = (p_v[sl] - t_v[sl]) * m
        acc_sq = acc_sq + d * d
        acc_ct = acc_ct + m

    part_v[0, :] = acc_sq
    part_v[1, :] = acc_ct
    pltpu.sync_copy(part_v, shared.at[wid])
    plsc.subcore_barrier()

    @pl.when(wid == 0)
    def _():
        pltpu.sync_copy(shared, all_v)
        tot_sq = jnp.zeros((L,), jnp.float32)
        tot_ct = jnp.zeros((L,), jnp.float32)
        for w in range(NS):
            tot_sq = tot_sq + all_v[w, 0, :]
            tot_ct = tot_ct + all_v[w, 1, :]

        idx = lax.iota(jnp.int32, L)

        def lane_sum(v):
            for sh in (1, 2, 4, 8):
                res_v[...] = v
                v = v + plsc.load_gather(res_v, [jnp.bitwise_xor(idx, sh)])
            return v

        sq = lane_sum(tot_sq)
        ct = lane_sum(tot_ct)
        res_v[...] = sq / ct
        pltpu.sync_copy(res_v, out_hbm)


@jax.jit
def kernel(predict, events, RFS_time):
    out = _masked_mse_sc(predict, events.astype(jnp.int32), RFS_time)
    return out[0]


# speedup vs baseline: 1.0062x; 1.0062x over previous
import functools

import jax
import jax.numpy as jnp
from jax import lax
from jax.experimental import pallas as pl
from jax.experimental.pallas import tpu as pltpu
from jax.experimental.pallas import tpu_sc as plsc

N = 16384
NS = 16
CHUNK = N // NS
L = 16
VECS = CHUNK // L

_mesh = plsc.VectorSubcoreMesh(
    core_axis_name="c", subcore_axis_name="s", num_cores=1, num_subcores=NS
)


@functools.partial(
    pl.kernel,
    out_type=jax.ShapeDtypeStruct((L,), jnp.float32),
    mesh=_mesh,
    compiler_params=pltpu.CompilerParams(
        needs_layout_passes=False, skip_device_barrier=True
    ),
    scratch_types=[
        pltpu.VMEM((CHUNK,), jnp.float32),
        pltpu.VMEM((CHUNK,), jnp.int32),
        pltpu.VMEM((CHUNK,), jnp.float32),
        pltpu.VMEM((2, L), jnp.float32),
        pltpu.VMEM((NS - 1, 2, L), jnp.float32),
        pltpu.VMEM((L,), jnp.float32),
        pltpu.VMEM_SHARED((NS, 2, L), jnp.float32),
        pltpu.SemaphoreType.DMA,
    ],
)
def _masked_mse_sc(predict_hbm, events_hbm, rfs_hbm, out_hbm,
                   p_v, e_v, t_v, part_v, all_v, res_v, shared, sem):
    wid = lax.axis_index("s")
    base = wid * CHUNK

    c1 = pltpu.make_async_copy(predict_hbm.at[pl.ds(base, CHUNK)], p_v, sem)
    c2 = pltpu.make_async_copy(events_hbm.at[pl.ds(base, CHUNK)], e_v, sem)
    c3 = pltpu.make_async_copy(rfs_hbm.at[pl.ds(base, CHUNK)], t_v, sem)
    c1.start()
    c2.start()
    c3.start()
    c1.wait()
    c2.wait()
    c3.wait()

    acc_sq = jnp.zeros((L,), jnp.float32)
    acc_ct = jnp.zeros((L,), jnp.float32)
    for i in range(VECS):
        sl = pl.ds(i * L, L)
        m = e_v[sl].astype(jnp.float32)
        d = (p_v[sl] - t_v[sl]) * m
        acc_sq = acc_sq + d * d
        acc_ct = acc_ct + m

    @pl.when(wid != 0)
    def _():
        part_v[0, :] = acc_sq
        part_v[1, :] = acc_ct
        pltpu.sync_copy(part_v, shared.at[wid])

    plsc.subcore_barrier()

    @pl.when(wid == 0)
    def _():
        pltpu.sync_copy(shared.at[pl.ds(1, NS - 1)], all_v)
        tot_sq = acc_sq
        tot_ct = acc_ct
        for w in range(NS - 1):
            tot_sq = tot_sq + all_v[w, 0, :]
            tot_ct = tot_ct + all_v[w, 1, :]

        idx = lax.iota(jnp.int32, L)
        sq, ct = tot_sq, tot_ct
        for sh in (1, 2, 4, 8):
            res_v[...] = sq
            part_v[0, :] = ct
            shuf = jnp.bitwise_xor(idx, sh)
            sq = sq + plsc.load_gather(res_v, [shuf])
            ct = ct + plsc.load_gather(part_v, [jnp.zeros((L,), jnp.int32), shuf])
        res_v[...] = sq / ct
        pltpu.sync_copy(res_v, out_hbm)


@jax.jit
def kernel(predict, events, RFS_time):
    out = _masked_mse_sc(predict, events.astype(jnp.int32), RFS_time)
    return out[0]
